# Initial kernel scaffold; baseline (speedup 1.0000x reference)
import jax
import jax.numpy as jnp
from jax.experimental import pallas as pl


def kernel(inputs, embed_weight):
    raise NotImplementedError("write your pallas kernel here")


# speedup vs baseline: 1.0826x; 1.0826x over previous
import functools

import jax
import jax.numpy as jnp
from jax import lax
from jax.experimental import pallas as pl
from jax.experimental.pallas import tpu as pltpu
from jax.experimental.pallas import tpu_sc as plsc

_N_EMBED = 8192
_EMBED_DIM = 256
_BETA = 0.25
_TOK = 8192

_T_TILE = 256
_C_TILE = 4096

_NC = 2
_NS = 16
_NW = _NC * _NS
_BPW = _TOK // _NW


def _round_bf16(v):
    return v.astype(jnp.bfloat16).astype(jnp.float32)


def _argmin_body(xb_ref, wb_ref, xsq_ref, wsq_ref, idx_ref, loss_ref,
                 vmin_ref, vidx_ref):
    j = pl.program_id(1)
    nj = pl.num_programs(1)
    m = lax.dot_general(xb_ref[...], wb_ref[...], (((1,), (1,)), ((), ())),
                        preferred_element_type=jnp.float32)
    dist = (xsq_ref[...] + wsq_ref[...]) - 2.0 * m
    tmin = jnp.min(dist, axis=1, keepdims=True)
    cols = lax.broadcasted_iota(jnp.int32, dist.shape, 1) + j * _C_TILE
    tidx = jnp.min(jnp.where(dist == tmin, cols, jnp.int32(2**30)),
                   axis=1, keepdims=True)

    @pl.when(j == 0)
    def _():
        vmin_ref[...] = _round_bf16(tmin)
        vidx_ref[...] = tidx

    @pl.when(j > 0)
    def _():
        better = tmin < vmin_ref[...]
        vmin_ref[...] = _round_bf16(jnp.where(better, tmin, vmin_ref[...]))
        vidx_ref[...] = jnp.where(better, tidx, vidx_ref[...])

    @pl.when(j == nj - 1)
    def _():
        idx_ref[...] = vidx_ref[...]
        s = jnp.sum(vmin_ref[...])
        i = pl.program_id(0)

        @pl.when(i == 0)
        def _():
            loss_ref[0, 0] = s

        @pl.when(i > 0)
        def _():
            loss_ref[0, 0] = loss_ref[0, 0] + s


def _tc_argmin(xb, wb, xsq, wsq):
    grid = (_TOK // _T_TILE, _N_EMBED // _C_TILE)
    return pl.pallas_call(
        _argmin_body,
        grid=grid,
        in_specs=[
            pl.BlockSpec((_T_TILE, _EMBED_DIM), lambda i, j: (i, 0)),
            pl.BlockSpec((_C_TILE, _EMBED_DIM), lambda i, j: (j, 0)),
            pl.BlockSpec((_T_TILE, 1), lambda i, j: (i, 0)),
            pl.BlockSpec((1, _C_TILE), lambda i, j: (0, j)),
        ],
        out_specs=[
            pl.BlockSpec((_T_TILE, 1), lambda i, j: (i, 0)),
            pl.BlockSpec((1, 1), lambda i, j: (0, 0),
                         memory_space=pltpu.SMEM),
        ],
        out_shape=[
            jax.ShapeDtypeStruct((_TOK, 1), jnp.int32),
            jax.ShapeDtypeStruct((1, 1), jnp.float32),
        ],
        scratch_shapes=[
            pltpu.VMEM((_T_TILE, 1), jnp.float32),
            pltpu.VMEM((_T_TILE, 1), jnp.int32),
        ],
    )(xb, wb, xsq, wsq)


def _gather_rows(table, idx):
    mesh = plsc.VectorSubcoreMesh(core_axis_name="c", subcore_axis_name="s")

    @functools.partial(
        pl.kernel,
        mesh=mesh,
        out_type=jax.ShapeDtypeStruct((_TOK, _EMBED_DIM), jnp.float32),
        scratch_types=[
            pltpu.VMEM((_BPW,), jnp.int32),
            pltpu.VMEM((_BPW, _EMBED_DIM), jnp.float32),
            pltpu.SemaphoreType.DMA,
        ],
    )
    def k(table_hbm, idx_hbm, out_hbm, idx_v, rows_v, sem):
        wid = lax.axis_index("s") * _NC + lax.axis_index("c")
        base = wid * _BPW
        pltpu.sync_copy(idx_hbm.at[pl.ds(base, _BPW)], idx_v)
        pltpu.async_copy(table_hbm.at[idx_v], rows_v, sem).wait()
        pltpu.sync_copy(rows_v, out_hbm.at[pl.ds(base, _BPW)])

    return k(table, idx)


def kernel(inputs, embed_weight):
    input_shape = inputs.shape
    flat = inputs.reshape(-1, _EMBED_DIM)
    xsq = jnp.sum(flat ** 2, axis=1, keepdims=True)
    wsq = jnp.sum(embed_weight ** 2, axis=1)[None, :]
    xb = flat.astype(jnp.bfloat16)
    wb = embed_weight.astype(jnp.bfloat16)
    idx2, loss_sum = _tc_argmin(xb, wb, xsq, wsq)
    idx = idx2.reshape(_TOK)
    quantized = _gather_rows(wb.astype(jnp.float32), idx)
    loss = loss_sum[0, 0] * ((1.0 + _BETA) / (_TOK * _EMBED_DIM))
    return (quantized.reshape(input_shape), loss,
            idx.reshape(input_shape[0], -1))


# speedup vs baseline: 1.0990x; 1.0152x over previous
import functools

import jax
import jax.numpy as jnp
from jax import lax
from jax.experimental import pallas as pl
from jax.experimental.pallas import tpu as pltpu
from jax.experimental.pallas import tpu_sc as plsc

_N_EMBED = 8192
_EMBED_DIM = 256
_BETA = 0.25
_TOK = 8192

_T_TILE = 512
_C_TILE = 4096

_NC = 2
_NS = 16
_NW = _NC * _NS
_BPW = _TOK // _NW


def _round_bf16(v):
    return v.astype(jnp.bfloat16).astype(jnp.float32)


def _argmin_body(xb_ref, wb_ref, xsq_ref, wsq_ref, idx_ref, loss_ref,
                 vmin_ref, vidx_ref):
    j = pl.program_id(1)
    nj = pl.num_programs(1)
    m2 = lax.dot_general(xb_ref[...], wb_ref[...], (((1,), (1,)), ((), ())),
                         preferred_element_type=jnp.float32)
    dist = (xsq_ref[...] + wsq_ref[...]) - m2
    tmin = jnp.min(dist, axis=1, keepdims=True)
    cols = lax.broadcasted_iota(jnp.int32, dist.shape, 1) + j * _C_TILE
    tidx = jnp.min(jnp.where(dist == tmin, cols, jnp.int32(2**30)),
                   axis=1, keepdims=True)

    @pl.when(j == 0)
    def _():
        vmin_ref[...] = _round_bf16(tmin)
        vidx_ref[...] = tidx

    @pl.when(j > 0)
    def _():
        better = tmin < vmin_ref[...]
        vmin_ref[...] = _round_bf16(jnp.where(better, tmin, vmin_ref[...]))
        vidx_ref[...] = jnp.where(better, tidx, vidx_ref[...])

    @pl.when(j == nj - 1)
    def _():
        idx_ref[...] = vidx_ref[...]
        s = jnp.sum(vmin_ref[...])
        i = pl.program_id(0)

        @pl.when(i == 0)
        def _():
            loss_ref[0, 0] = s

        @pl.when(i > 0)
        def _():
            loss_ref[0, 0] = loss_ref[0, 0] + s


def _tc_argmin(xb, wb, xsq, wsq):
    grid = (_TOK // _T_TILE, _N_EMBED // _C_TILE)
    return pl.pallas_call(
        _argmin_body,
        grid=grid,
        in_specs=[
            pl.BlockSpec((_T_TILE, _EMBED_DIM), lambda i, j: (i, 0)),
            pl.BlockSpec((_C_TILE, _EMBED_DIM), lambda i, j: (j, 0)),
            pl.BlockSpec((_T_TILE, 1), lambda i, j: (i, 0)),
            pl.BlockSpec((1, _C_TILE), lambda i, j: (0, j)),
        ],
        out_specs=[
            pl.BlockSpec((_T_TILE, 1), lambda i, j: (i, 0)),
            pl.BlockSpec((1, 1), lambda i, j: (0, 0),
                         memory_space=pltpu.SMEM),
        ],
        out_shape=[
            jax.ShapeDtypeStruct((_TOK, 1), jnp.int32),
            jax.ShapeDtypeStruct((1, 1), jnp.float32),
        ],
        scratch_shapes=[
            pltpu.VMEM((_T_TILE, 1), jnp.float32),
            pltpu.VMEM((_T_TILE, 1), jnp.int32),
        ],
    )(xb, wb, xsq, wsq)


def _gather_rows(table, idx):
    mesh = plsc.VectorSubcoreMesh(core_axis_name="c", subcore_axis_name="s")

    @functools.partial(
        pl.kernel,
        mesh=mesh,
        out_type=jax.ShapeDtypeStruct((_TOK, _EMBED_DIM), jnp.float32),
        scratch_types=[
            pltpu.VMEM((_BPW,), jnp.int32),
            pltpu.VMEM((_BPW, _EMBED_DIM), jnp.float32),
            pltpu.SemaphoreType.DMA,
        ],
    )
    def k(table_hbm, idx_hbm, out_hbm, idx_v, rows_v, sem):
        wid = lax.axis_index("s") * _NC + lax.axis_index("c")
        base = wid * _BPW
        pltpu.sync_copy(idx_hbm.at[pl.ds(base, _BPW)], idx_v)
        pltpu.async_copy(table_hbm.at[idx_v], rows_v, sem).wait()
        pltpu.sync_copy(rows_v, out_hbm.at[pl.ds(base, _BPW)])

    return k(table, idx)


def kernel(inputs, embed_weight):
    input_shape = inputs.shape
    flat = inputs.reshape(-1, _EMBED_DIM)
    xsq = jnp.sum(flat ** 2, axis=1, keepdims=True)
    wsq = jnp.sum(embed_weight ** 2, axis=1)[None, :]
    xb = (flat + flat).astype(jnp.bfloat16)
    wb = embed_weight.astype(jnp.bfloat16)
    idx2, loss_sum = _tc_argmin(xb, wb, xsq, wsq)
    idx = idx2.reshape(_TOK)
    quantized = _gather_rows(wb.astype(jnp.float32), idx)
    loss = loss_sum[0, 0] * ((1.0 + _BETA) / (_TOK * _EMBED_DIM))
    return (quantized.reshape(input_shape), loss,
            idx.reshape(input_shape[0], -1))


# speedup vs baseline: 1.1145x; 1.0141x over previous
import functools

import jax
import jax.numpy as jnp
from jax import lax
from jax.experimental import pallas as pl
from jax.experimental.pallas import tpu as pltpu
from jax.experimental.pallas import tpu_sc as plsc

_N_EMBED = 8192
_EMBED_DIM = 256
_BETA = 0.25
_TOK = 8192

_T_TILE = 1024
_C_TILE = 4096

_NC = 2
_NS = 16
_NW = _NC * _NS
_BPW = _TOK // _NW


def _round_bf16(v):
    return v.astype(jnp.bfloat16).astype(jnp.float32)


def _argmin_body(xb_ref, wb_ref, xsq_ref, wsq_ref, idx_ref, loss_ref,
                 vmin_ref, vidx_ref):
    j = pl.program_id(1)
    nj = pl.num_programs(1)
    m2 = lax.dot_general(xb_ref[...], wb_ref[...], (((1,), (0,)), ((), ())),
                         preferred_element_type=jnp.float32)
    dist = (xsq_ref[...] + wsq_ref[...]) - m2
    tmin = jnp.min(dist, axis=1, keepdims=True)
    cols = lax.broadcasted_iota(jnp.int32, dist.shape, 1) + j * _C_TILE
    tidx = jnp.min(jnp.where(dist == tmin, cols, jnp.int32(2**30)),
                   axis=1, keepdims=True)

    @pl.when(j == 0)
    def _():
        vmin_ref[...] = _round_bf16(tmin)
        vidx_ref[...] = tidx

    @pl.when(j > 0)
    def _():
        better = tmin < vmin_ref[...]
        vmin_ref[...] = _round_bf16(jnp.where(better, tmin, vmin_ref[...]))
        vidx_ref[...] = jnp.where(better, tidx, vidx_ref[...])

    @pl.when(j == nj - 1)
    def _():
        idx_ref[...] = vidx_ref[...]
        s = jnp.sum(vmin_ref[...])
        i = pl.program_id(0)

        @pl.when(i == 0)
        def _():
            loss_ref[0, 0] = s

        @pl.when(i > 0)
        def _():
            loss_ref[0, 0] = loss_ref[0, 0] + s


def _tc_argmin(xb, wb, xsq, wsq):
    grid = (_TOK // _T_TILE, _N_EMBED // _C_TILE)
    return pl.pallas_call(
        _argmin_body,
        grid=grid,
        in_specs=[
            pl.BlockSpec((_T_TILE, _EMBED_DIM), lambda i, j: (i, 0)),
            pl.BlockSpec((_EMBED_DIM, _C_TILE), lambda i, j: (0, j)),
            pl.BlockSpec((_T_TILE, 1), lambda i, j: (i, 0)),
            pl.BlockSpec((1, _C_TILE), lambda i, j: (0, j)),
        ],
        out_specs=[
            pl.BlockSpec((_T_TILE, 1), lambda i, j: (i, 0)),
            pl.BlockSpec((1, 1), lambda i, j: (0, 0),
                         memory_space=pltpu.SMEM),
        ],
        out_shape=[
            jax.ShapeDtypeStruct((_TOK, 1), jnp.int32),
            jax.ShapeDtypeStruct((1, 1), jnp.float32),
        ],
        scratch_shapes=[
            pltpu.VMEM((_T_TILE, 1), jnp.float32),
            pltpu.VMEM((_T_TILE, 1), jnp.int32),
        ],
    )(xb, wb, xsq, wsq)


def _gather_rows(table, idx):
    mesh = plsc.VectorSubcoreMesh(core_axis_name="c", subcore_axis_name="s")

    @functools.partial(
        pl.kernel,
        mesh=mesh,
        out_type=jax.ShapeDtypeStruct((_TOK, _EMBED_DIM), jnp.float32),
        scratch_types=[
            pltpu.VMEM((_BPW,), jnp.int32),
            pltpu.VMEM((_BPW, _EMBED_DIM), jnp.float32),
            pltpu.SemaphoreType.DMA,
        ],
    )
    def k(table_hbm, idx_hbm, out_hbm, idx_v, rows_v, sem):
        wid = lax.axis_index("s") * _NC + lax.axis_index("c")
        base = wid * _BPW
        pltpu.sync_copy(idx_hbm.at[pl.ds(base, _BPW)], idx_v)
        pltpu.async_copy(table_hbm.at[idx_v], rows_v, sem).wait()
        pltpu.sync_copy(rows_v, out_hbm.at[pl.ds(base, _BPW)])

    return k(table, idx)


def kernel(inputs, embed_weight):
    input_shape = inputs.shape
    flat = inputs.reshape(-1, _EMBED_DIM)
    xsq = jnp.sum(flat ** 2, axis=1, keepdims=True)
    wsq = jnp.sum(embed_weight ** 2, axis=1)[None, :]
    xb = (flat + flat).astype(jnp.bfloat16)
    wb = embed_weight.astype(jnp.bfloat16)
    idx2, loss_sum = _tc_argmin(xb, wb.T, xsq, wsq)
    idx = idx2.reshape(_TOK)
    quantized = _gather_rows(wb.astype(jnp.float32), idx)

    loss = loss_sum[0, 0] * ((1.0 + _BETA) / (_TOK * _EMBED_DIM))
    return (quantized.reshape(input_shape), loss,
            idx.reshape(input_shape[0], -1))


# speedup vs baseline: 1.1928x; 1.0703x over previous
import functools

import jax
import jax.numpy as jnp
from jax import lax
from jax.experimental import pallas as pl
from jax.experimental.pallas import tpu as pltpu
from jax.experimental.pallas import tpu_sc as plsc

_N_EMBED = 8192
_EMBED_DIM = 256
_BETA = 0.25
_TOK = 8192

_T_TILE = 1024
_C_TILE = 4096

_NC = 2
_NS = 16
_NW = _NC * _NS
_BPW = _TOK // _NW


def _round_bf16(v):
    return v.astype(jnp.bfloat16).astype(jnp.float32)


def _argmin_body(xb_ref, wb_ref, xsq_ref, wsq_ref, cols_ref, idx_ref,
                 loss_ref, vmin_ref, vidx_ref):
    j = pl.program_id(1)
    nj = pl.num_programs(1)
    m2 = lax.dot_general(xb_ref[...], wb_ref[...], (((1,), (0,)), ((), ())),
                         preferred_element_type=jnp.float32)
    dist = (xsq_ref[...] + wsq_ref[...]) - m2
    tmin = jnp.min(dist, axis=1, keepdims=True)
    tidx_f = jnp.min(jnp.where(dist == tmin, cols_ref[...], jnp.float32(2**24)),
                     axis=1, keepdims=True)
    tidx = tidx_f.astype(jnp.int32)

    @pl.when(j == 0)
    def _():
        vmin_ref[...] = _round_bf16(tmin)
        vidx_ref[...] = tidx

    @pl.when(j > 0)
    def _():
        better = tmin < vmin_ref[...]
        vmin_ref[...] = _round_bf16(jnp.where(better, tmin, vmin_ref[...]))
        vidx_ref[...] = jnp.where(better, tidx, vidx_ref[...])

    @pl.when(j == nj - 1)
    def _():
        idx_ref[...] = vidx_ref[...]
        s = jnp.sum(vmin_ref[...])
        i = pl.program_id(0)

        @pl.when(i == 0)
        def _():
            loss_ref[0, 0] = s

        @pl.when(i > 0)
        def _():
            loss_ref[0, 0] = loss_ref[0, 0] + s


def _tc_argmin(xb, wb, xsq, wsq, cols):
    grid = (_TOK // _T_TILE, _N_EMBED // _C_TILE)
    return pl.pallas_call(
        _argmin_body,
        grid=grid,
        in_specs=[
            pl.BlockSpec((_T_TILE, _EMBED_DIM), lambda i, j: (i, 0)),
            pl.BlockSpec((_EMBED_DIM, _C_TILE), lambda i, j: (0, j)),
            pl.BlockSpec((_T_TILE, 1), lambda i, j: (i, 0)),
            pl.BlockSpec((1, _C_TILE), lambda i, j: (0, j)),
            pl.BlockSpec((1, _C_TILE), lambda i, j: (0, j)),
        ],
        out_specs=[
            pl.BlockSpec((_T_TILE, 1), lambda i, j: (i, 0)),
            pl.BlockSpec((1, 1), lambda i, j: (0, 0),
                         memory_space=pltpu.SMEM),
        ],
        out_shape=[
            jax.ShapeDtypeStruct((_TOK, 1), jnp.int32),
            jax.ShapeDtypeStruct((1, 1), jnp.float32),
        ],
        scratch_shapes=[
            pltpu.VMEM((_T_TILE, 1), jnp.float32),
            pltpu.VMEM((_T_TILE, 1), jnp.int32),
        ],
    )(xb, wb, xsq, wsq, cols)


def _gather_rows(table, idx):
    mesh = plsc.VectorSubcoreMesh(core_axis_name="c", subcore_axis_name="s")

    @functools.partial(
        pl.kernel,
        mesh=mesh,
        out_type=jax.ShapeDtypeStruct((_TOK, _EMBED_DIM), jnp.float32),
        scratch_types=[
            pltpu.VMEM((_BPW,), jnp.int32),
            pltpu.VMEM((_BPW, _EMBED_DIM), jnp.float32),
            pltpu.SemaphoreType.DMA,
        ],
    )
    def k(table_hbm, idx_hbm, out_hbm, idx_v, rows_v, sem):
        wid = lax.axis_index("s") * _NC + lax.axis_index("c")
        base = wid * _BPW
        pltpu.sync_copy(idx_hbm.at[pl.ds(base, _BPW)], idx_v)
        pltpu.async_copy(table_hbm.at[idx_v], rows_v, sem).wait()
        pltpu.sync_copy(rows_v, out_hbm.at[pl.ds(base, _BPW)])

    return k(table, idx)


def kernel(inputs, embed_weight):
    input_shape = inputs.shape
    flat = inputs.reshape(-1, _EMBED_DIM)
    xsq = jnp.sum(flat ** 2, axis=1, keepdims=True)
    wsq = jnp.sum(embed_weight ** 2, axis=1)[None, :]
    xb = (flat + flat).astype(jnp.bfloat16)
    wb = embed_weight.astype(jnp.bfloat16)
    cols = jnp.arange(_N_EMBED, dtype=jnp.float32)[None, :]
    idx2, loss_sum = _tc_argmin(xb, wb.T, xsq, wsq, cols)
    idx = idx2.reshape(_TOK)
    quantized = _gather_rows(wb.astype(jnp.float32), idx)

    loss = loss_sum[0, 0] * ((1.0 + _BETA) / (_TOK * _EMBED_DIM))
    return (quantized.reshape(input_shape), loss,
            idx.reshape(input_shape[0], -1))
